# baseline (speedup 1.0000x reference)
import jax
import jax.numpy as jnp
from jax import lax
from jax.experimental import pallas as pl
from jax.experimental.pallas import tpu as pltpu
from jax.experimental.pallas import tpu_sc as plsc

NC, NS = 2, 16
NW = NC * NS
NE = 1_000_000
NR = 26
H, M, B = 2, 32, 1024
UNITS = H * M * (B // 128)
UPW = UNITS // NW

_cache = []


def _body(et_hbm, rt_hbm, hphi_hbm, tphi_hbm, rphi_hbm, v_hbm,
          hsp_hbm, tsp_hbm, rsp_hbm, vsp_hbm,
          relT_v, hidx_v, tidx_v, ridx_v, r26_v, fidx_v, tile_v, rtile_v,
          sem, gsem):
    wid = lax.axis_index("s") * NC + lax.axis_index("c")
    iota = lax.broadcasted_iota(jnp.int32, (16,), 0)

    pltpu.sync_copy(rt_hbm, relT_v)

    def ent_tile(eidx_ref, dst_ref, i):
        for r in range(8):
            base = (8 * i + r) * NE
            for g in range(8):
                fidx_v[r, pl.ds(g * 16, 16)] = eidx_ref[pl.ds(g * 16, 16)] + base
        for r in range(8):
            pltpu.async_copy(et_hbm.at[fidx_v.at[r]], tile_v.at[r], gsem)
        for r in range(8):
            pltpu.make_async_copy(et_hbm.at[fidx_v.at[r]], tile_v.at[r], gsem).wait()
        pltpu.sync_copy(tile_v, dst_ref)

    def unit_body(u, _):
        uid = wid * UPW + u
        h = uid // (M * 8)
        m = (uid // 8) % M
        j = uid % 8

        pltpu.sync_copy(hphi_hbm.at[h, m, pl.ds(j * 128, 128)], hidx_v)
        pltpu.sync_copy(tphi_hbm.at[h, m, pl.ds(j * 128, 128)], tidx_v)
        pltpu.sync_copy(rphi_hbm.at[h, m, pl.ds(j * 128, 128)], ridx_v)

        for i in range(2):
            ent_tile(hidx_v, hsp_hbm.at[h, m, i, j], i)
            ent_tile(tidx_v, tsp_hbm.at[h, m, i, j], i)

        def d1_body(d1, _):
            for i in range(2):
                for r in range(8):
                    row = d1 * 16 + 8 * i + r
                    for g in range(8):
                        idx = ridx_v[pl.ds(g * 16, 16)] + row * NR
                        rtile_v[r, pl.ds(g * 16, 16)] = plsc.load_gather(
                            relT_v, [idx])
                pltpu.sync_copy(rtile_v, rsp_hbm.at[h, m, d1, i, j])
            return 0

        lax.fori_loop(0, 16, d1_body, 0, unroll=False)
        return 0

    lax.fori_loop(0, UPW, unit_body, 0, unroll=False)

    @pl.when(wid < 16)
    def _():
        i = wid // 8
        j = wid % 8
        pltpu.sync_copy(v_hbm.at[pl.ds(j * 128, 128)], hidx_v)
        ent_tile(hidx_v, vsp_hbm.at[i, j], i)


def _call():
    if not _cache:
        mesh = plsc.VectorSubcoreMesh(core_axis_name="c", subcore_axis_name="s",
                                      num_cores=NC, num_subcores=NS)
        _cache.append(pl.kernel(
            _body,
            out_type=(
                jax.ShapeDtypeStruct((H, M, 2, 8, 8, 128), jnp.float32),
                jax.ShapeDtypeStruct((H, M, 2, 8, 8, 128), jnp.float32),
                jax.ShapeDtypeStruct((H, M, 16, 2, 8, 8, 128), jnp.float32),
                jax.ShapeDtypeStruct((2, 8, 8, 128), jnp.float32),
            ),
            mesh=mesh,
            scratch_types=[
                pltpu.VMEM((256 * NR,), jnp.float32),
                pltpu.VMEM((128,), jnp.int32),
                pltpu.VMEM((128,), jnp.int32),
                pltpu.VMEM((128,), jnp.int32),
                pltpu.VMEM((128,), jnp.int32),
                pltpu.VMEM((8, 128), jnp.int32),
                pltpu.VMEM((8, 128), jnp.float32),
                pltpu.VMEM((8, 128), jnp.float32),
                pltpu.SemaphoreType.DMA,
                pltpu.SemaphoreType.DMA,
            ],
            compiler_params=pltpu.CompilerParams(needs_layout_passes=False),
        ))
    return _cache[0]


def kernel(h_i, R_i, t_i, v_i, entity_emb, relation_emb):
    eT = entity_emb.T.reshape(-1)
    relT = relation_emb.reshape(NR, 256).T.reshape(-1)
    hphi = jnp.transpose(h_i, (1, 2, 0))
    tphi = jnp.transpose(t_i, (1, 2, 0))
    rphi = jnp.transpose(R_i, (1, 2, 0))

    hsp, tsp, rsp, vsp = _call()(eT, relT, hphi, tphi, rphi, v_i)

    hs = (hsp.transpose(0, 1, 3, 5, 2, 4)
             .reshape(H, M, B, 16)
             .transpose(2, 0, 1, 3))
    ts = (tsp.transpose(0, 1, 3, 5, 2, 4)
             .reshape(H, M, B, 16)
             .transpose(2, 0, 1, 3))
    Rs = (rsp.transpose(0, 1, 2, 4, 6, 3, 5)
             .reshape(H, M, 16, B, 16)
             .transpose(3, 0, 1, 2, 4))
    vs = (vsp.transpose(1, 3, 0, 2)
             .reshape(B, 16))
    return (hs, Rs, ts, vs)


# speedup vs baseline: 1.7161x; 1.7161x over previous
import jax
import jax.numpy as jnp
from jax import lax
from jax.experimental import pallas as pl
from jax.experimental.pallas import tpu as pltpu
from jax.experimental.pallas import tpu_sc as plsc

NC, NS = 2, 16
NW = NC * NS
NE = 1_000_000
NR = 26
H, M, B = 2, 32, 1024

_cache = {}


_DT_COLS = 4096
_DT_GRID = (NE + _DT_COLS - 1) // _DT_COLS


def _detile_body(in_ref, out_ref):
    x = in_ref[...]
    x = x.reshape(16, _DT_COLS // 8, 8)
    x = x.transpose(1, 2, 0)
    out_ref[...] = x.reshape(_DT_COLS // 8, 128)


def _detile():
    if "dt" not in _cache:
        _cache["dt"] = pl.pallas_call(
            _detile_body,
            grid=(_DT_GRID,),
            in_specs=[pl.BlockSpec((16, _DT_COLS), lambda t: (0, t))],
            out_specs=pl.BlockSpec((_DT_COLS // 8, 128), lambda t: (t, 0)),
            out_shape=jax.ShapeDtypeStruct((125000, 128), jnp.float32),
        )
    return _cache["dt"]


def _rs_body(relT_ref, ridx_ref, out_ref):
    r = ridx_ref[0, 0, 0, :]
    onehot = (lax.broadcasted_iota(jnp.int32, (NR, B), 0)
              == r[None, :]).astype(jnp.float32)
    out_ref[0, 0] = jnp.dot(relT_ref[...], onehot,
                            preferred_element_type=jnp.float32,
                            precision=lax.Precision.HIGHEST)


def _rs_call():
    if "rs" not in _cache:
        _cache["rs"] = pl.pallas_call(
            _rs_body,
            grid=(H, M),
            in_specs=[
                pl.BlockSpec((256, NR), lambda h, m: (0, 0)),
                pl.BlockSpec((1, 1, 1, B), lambda h, m: (h, m, 0, 0)),
            ],
            out_specs=pl.BlockSpec((1, 1, 256, B), lambda h, m: (h, m, 0, 0)),
            out_shape=jax.ShapeDtypeStruct((H, M, 256, B), jnp.float32),
        )
    return _cache["rs"]


def _sc_body(er_hbm, hphi_hbm, tphi_hbm, v_hbm,
             hsp_hbm, tsp_hbm, vsp_hbm,
             eidx_v, hrows_v, trows_v, gvec_v, tile_v, sem, gsem):
    wid = lax.axis_index("s") * NC + lax.axis_index("c")
    iota = lax.broadcasted_iota(jnp.int32, (16,), 0)
    h = wid // 16
    m2 = 2 * (wid % 16)

    def transpose_tiles(rows_ref, out_hbm, m):
        def j_body(j, _):
            for g in range(8):
                gvec_v[pl.ds(g * 16, 16)] = iota + (j * 128 + g * 16)
            for i in range(2):
                for ds in range(8):
                    d = 8 * i + ds
                    dvec = jnp.full((16,), d, jnp.int32)
                    for g in range(8):
                        bvec = gvec_v[pl.ds(g * 16, 16)]
                        tile_v[ds, pl.ds(g * 16, 16)] = plsc.load_gather(
                            rows_ref, [bvec, dvec])
                pltpu.sync_copy(tile_v, out_hbm.at[h, m, i, j])
            return 0
        lax.fori_loop(0, 8, j_body, 0, unroll=False)

    for mm in range(2):
        m = m2 + mm
        pltpu.sync_copy(hphi_hbm.at[h, m], eidx_v)
        pltpu.async_copy(er_hbm.at[eidx_v], hrows_v, gsem).wait()
        pltpu.sync_copy(tphi_hbm.at[h, m], eidx_v)
        pltpu.async_copy(er_hbm.at[eidx_v], trows_v, gsem).wait()
        transpose_tiles(hrows_v, hsp_hbm, m)
        transpose_tiles(trows_v, tsp_hbm, m)

    @pl.when(wid < 16)
    def _():
        i = wid // 8
        j = wid % 8
        pltpu.sync_copy(v_hbm.at[pl.ds(j * 128, 128)], eidx_v.at[pl.ds(0, 128)])
        pltpu.async_copy(er_hbm.at[eidx_v.at[pl.ds(0, 128)]],
                         hrows_v.at[pl.ds(0, 128)], gsem).wait()
        for g in range(8):
            gvec_v[pl.ds(g * 16, 16)] = iota + g * 16
        for ds in range(8):
            d = 8 * i + ds
            dvec = jnp.full((16,), d, jnp.int32)
            for g in range(8):
                bvec = gvec_v[pl.ds(g * 16, 16)]
                tile_v[ds, pl.ds(g * 16, 16)] = plsc.load_gather(
                    hrows_v, [bvec, dvec])
        pltpu.sync_copy(tile_v, vsp_hbm.at[i, j])


def _sc_call():
    if "sc" not in _cache:
        mesh = plsc.VectorSubcoreMesh(core_axis_name="c", subcore_axis_name="s",
                                      num_cores=NC, num_subcores=NS)
        _cache["sc"] = pl.kernel(
            _sc_body,
            out_type=(
                jax.ShapeDtypeStruct((H, M, 2, 8, 8, 128), jnp.float32),
                jax.ShapeDtypeStruct((H, M, 2, 8, 8, 128), jnp.float32),
                jax.ShapeDtypeStruct((2, 8, 8, 128), jnp.float32),
            ),
            mesh=mesh,
            scratch_types=[
                pltpu.VMEM((B,), jnp.int32),
                pltpu.VMEM((B, 16), jnp.float32),
                pltpu.VMEM((B, 16), jnp.float32),
                pltpu.VMEM((128,), jnp.int32),
                pltpu.VMEM((8, 128), jnp.float32),
                pltpu.SemaphoreType.DMA,
                pltpu.SemaphoreType.DMA,
            ],
            compiler_params=pltpu.CompilerParams(needs_layout_passes=False,
                                                 use_tc_tiling_on_sc=False),
        )
    return _cache["sc"]


def kernel(h_i, R_i, t_i, v_i, entity_emb, relation_emb):
    eT = entity_emb.T
    relT = relation_emb.reshape(NR, 256).T
    hphi = jnp.transpose(h_i, (1, 2, 0))
    tphi = jnp.transpose(t_i, (1, 2, 0))
    rphi = jnp.transpose(R_i, (1, 2, 0))

    er = _detile()(eT).reshape(NE, 16)
    rsp = _rs_call()(relT, rphi.reshape(H, M, 1, B))
    hsp, tsp, vsp = _sc_call()(er, hphi, tphi, v_i)

    hs = (hsp.transpose(0, 1, 3, 5, 2, 4)
             .reshape(H, M, B, 16)
             .transpose(2, 0, 1, 3))
    ts = (tsp.transpose(0, 1, 3, 5, 2, 4)
             .reshape(H, M, B, 16)
             .transpose(2, 0, 1, 3))
    Rs = (rsp.reshape(H, M, 16, 16, B)
             .transpose(4, 0, 1, 2, 3))
    vs = (vsp.transpose(1, 3, 0, 2)
             .reshape(B, 16))
    return (hs, Rs, ts, vs)


# speedup vs baseline: 7.6610x; 4.4642x over previous
import jax
import jax.numpy as jnp
from jax import lax
from jax.experimental import pallas as pl
from jax.experimental.pallas import tpu as pltpu
from jax.experimental.pallas import tpu_sc as plsc

NC, NS = 2, 16
NW = NC * NS
NE = 1_000_000
NR = 26
H, M, B = 2, 32, 1024

_cache = {}


def _detile_body(in_ref, out_ref):
    x = in_ref[...]
    out_ref[...] = x.reshape(8, 601, 128).transpose(1, 0, 2)[None]


def _detile():
    if "dt" not in _cache:
        _cache["dt"] = pl.pallas_call(
            _detile_body,
            grid=(2, 13),
            in_specs=[pl.BlockSpec((8, 76928), lambda i, c: (i, c))],
            out_specs=pl.BlockSpec((1, 601, 8, 128), lambda i, c: (i, c, 0, 0)),
            out_shape=jax.ShapeDtypeStruct((2, 7813, 8, 128), jnp.float32),
        )
    return _cache["dt"]


def _rs_body(relT_ref, ridx_ref, out_ref):
    r = ridx_ref[0, 0, 0, :]
    onehot = (lax.broadcasted_iota(jnp.int32, (NR, B), 0)
              == r[None, :]).astype(jnp.float32)
    out_ref[0, 0] = jnp.dot(relT_ref[...], onehot,
                            preferred_element_type=jnp.float32,
                            precision=lax.Precision.HIGHEST)


def _rs_call():
    if "rs" not in _cache:
        _cache["rs"] = pl.pallas_call(
            _rs_body,
            grid=(H, M),
            in_specs=[
                pl.BlockSpec((256, NR), lambda h, m: (0, 0)),
                pl.BlockSpec((1, 1, 1, B), lambda h, m: (h, m, 0, 0)),
            ],
            out_specs=pl.BlockSpec((1, 1, 256, B), lambda h, m: (h, m, 0, 0)),
            out_shape=jax.ShapeDtypeStruct((H, M, 256, B), jnp.float32),
        )
    return _cache["rs"]


def _sc_body(ef_hbm, hphi_hbm, tphi_hbm, v_hbm,
             hsp_hbm, tsp_hbm, vsp_hbm,
             eidx_v, pidx_v, fidx_v, tile_v, gsem, osem):
    wid = lax.axis_index("s") * NC + lax.axis_index("c")
    h = wid // 16
    m2 = 2 * (wid % 16)

    def build_pidx(n):
        for g in range(n):
            e = eidx_v[pl.ds(g * 16, 16)]
            pidx_v[pl.ds(g * 16, 16)] = ((e >> 7) << 10) + (e & 127)

    def ent_tiles(out_hbm, m):
        def j_body(j, _):
            for i in range(2):
                for ds in range(8):
                    base = i * 8000512 + ds * 128
                    for g in range(8):
                        pv = pidx_v[pl.ds(j * 128 + g * 16, 16)]
                        fidx_v[i, ds, pl.ds(g * 16, 16)] = pv + base
            for i in range(2):
                for ds in range(8):
                    pltpu.async_copy(ef_hbm.at[fidx_v.at[i, ds]],
                                     tile_v.at[i, ds], gsem)
            for i in range(2):
                for ds in range(8):
                    pltpu.make_async_copy(ef_hbm.at[fidx_v.at[i, ds]],
                                          tile_v.at[i, ds], gsem).wait()
            for i in range(2):
                pltpu.async_copy(tile_v.at[i], out_hbm.at[h, m, i, j], osem)
            for i in range(2):
                pltpu.make_async_copy(tile_v.at[i], out_hbm.at[h, m, i, j],
                                      osem).wait()
            return 0
        lax.fori_loop(0, 8, j_body, 0, unroll=False)

    for mm in range(2):
        m = m2 + mm
        pltpu.sync_copy(hphi_hbm.at[h, m], eidx_v)
        build_pidx(64)
        ent_tiles(hsp_hbm, m)
        pltpu.sync_copy(tphi_hbm.at[h, m], eidx_v)
        build_pidx(64)
        ent_tiles(tsp_hbm, m)

    @pl.when(wid < 16)
    def _():
        i = wid // 8
        j = wid % 8
        pltpu.sync_copy(v_hbm.at[pl.ds(j * 128, 128)], eidx_v.at[pl.ds(0, 128)])
        build_pidx(8)
        for ds in range(8):
            base = i * 8000512 + ds * 128
            for g in range(8):
                pv = pidx_v[pl.ds(g * 16, 16)]
                fidx_v[0, ds, pl.ds(g * 16, 16)] = pv + base
        for ds in range(8):
            pltpu.async_copy(ef_hbm.at[fidx_v.at[0, ds]], tile_v.at[0, ds], gsem)
        for ds in range(8):
            pltpu.make_async_copy(ef_hbm.at[fidx_v.at[0, ds]],
                                  tile_v.at[0, ds], gsem).wait()
        pltpu.sync_copy(tile_v.at[0], vsp_hbm.at[i, j])


def _sc_call():
    if "sc" not in _cache:
        mesh = plsc.VectorSubcoreMesh(core_axis_name="c", subcore_axis_name="s",
                                      num_cores=NC, num_subcores=NS)
        _cache["sc"] = pl.kernel(
            _sc_body,
            out_type=(
                jax.ShapeDtypeStruct((H, M, 2, 8, 8, 128), jnp.float32),
                jax.ShapeDtypeStruct((H, M, 2, 8, 8, 128), jnp.float32),
                jax.ShapeDtypeStruct((2, 8, 8, 128), jnp.float32),
            ),
            mesh=mesh,
            scratch_types=[
                pltpu.VMEM((B,), jnp.int32),
                pltpu.VMEM((B,), jnp.int32),
                pltpu.VMEM((2, 8, 128), jnp.int32),
                pltpu.VMEM((2, 8, 128), jnp.float32),
                pltpu.SemaphoreType.DMA,
                pltpu.SemaphoreType.DMA,
            ],
            compiler_params=pltpu.CompilerParams(needs_layout_passes=False,
                                                 use_tc_tiling_on_sc=False),
        )
    return _cache["sc"]


def kernel(h_i, R_i, t_i, v_i, entity_emb, relation_emb):
    eT = entity_emb.T
    relT = relation_emb.reshape(NR, 256).T
    hphi = jnp.transpose(h_i, (1, 2, 0))
    tphi = jnp.transpose(t_i, (1, 2, 0))
    rphi = jnp.transpose(R_i, (1, 2, 0))

    ef = _detile()(eT).reshape(-1)
    rsp = _rs_call()(relT, rphi.reshape(H, M, 1, B))
    hsp, tsp, vsp = _sc_call()(ef, hphi, tphi, v_i)

    hs = (hsp.transpose(0, 1, 3, 5, 2, 4)
             .reshape(H, M, B, 16)
             .transpose(2, 0, 1, 3))
    ts = (tsp.transpose(0, 1, 3, 5, 2, 4)
             .reshape(H, M, B, 16)
             .transpose(2, 0, 1, 3))
    Rs = (rsp.reshape(H, M, 16, 16, B)
             .transpose(4, 0, 1, 2, 3))
    vs = (vsp.transpose(1, 3, 0, 2)
             .reshape(B, 16))
    return (hs, Rs, ts, vs)


# speedup vs baseline: 8.6170x; 1.1248x over previous
import jax
import jax.numpy as jnp
from jax import lax
from jax.experimental import pallas as pl
from jax.experimental.pallas import tpu as pltpu
from jax.experimental.pallas import tpu_sc as plsc

NC, NS = 2, 16
NW = NC * NS
NE = 1_000_000
NR = 26
H, M, B = 2, 32, 1024

_cache = {}


def _detile_body(in_ref, out_ref):
    x = in_ref[...]
    out_ref[...] = x.reshape(8, 601, 128).transpose(1, 0, 2)[None]


def _detile():
    if "dt" not in _cache:
        _cache["dt"] = pl.pallas_call(
            _detile_body,
            grid=(2, 13),
            in_specs=[pl.BlockSpec((8, 76928), lambda i, c: (i, c))],
            out_specs=pl.BlockSpec((1, 601, 8, 128), lambda i, c: (i, c, 0, 0)),
            out_shape=jax.ShapeDtypeStruct((2, 7813, 8, 128), jnp.float32),
        )
    return _cache["dt"]


def _rs_body(relT_ref, ridx_ref, out_ref):
    r = ridx_ref[0, 0, 0, :]
    onehot = (lax.broadcasted_iota(jnp.int32, (NR, B), 0)
              == r[None, :]).astype(jnp.float32)
    out_ref[0, 0] = jnp.dot(relT_ref[...], onehot,
                            preferred_element_type=jnp.float32,
                            precision=lax.Precision.HIGHEST)


def _rs_call():
    if "rs" not in _cache:
        _cache["rs"] = pl.pallas_call(
            _rs_body,
            grid=(H, M),
            in_specs=[
                pl.BlockSpec((256, NR), lambda h, m: (0, 0)),
                pl.BlockSpec((1, 1, 1, B), lambda h, m: (h, m, 0, 0)),
            ],
            out_specs=pl.BlockSpec((1, 1, 256, B), lambda h, m: (h, m, 0, 0)),
            out_shape=jax.ShapeDtypeStruct((H, M, 256, B), jnp.float32),
        )
    return _cache["rs"]


def _sc_body(ef_hbm, hphi_hbm, tphi_hbm, v_hbm,
             hsp_hbm, tsp_hbm, vsp_hbm,
             eidx_v, pidx_v, fidx_v, tile_v, gsem, osem):
    wid = lax.axis_index("s") * NC + lax.axis_index("c")
    h = wid // 16
    m2 = 2 * (wid % 16)

    def build_pidx(n):
        for g in range(n):
            e = eidx_v[pl.ds(g * 16, 16)]
            pidx_v[pl.ds(g * 16, 16)] = ((e >> 7) << 10) + (e & 127)

    def build_fidx(j, buf):
        for i in range(2):
            for ds in range(8):
                base = i * 8000512 + ds * 128
                for g in range(8):
                    pv = pidx_v[pl.ds(j * 128 + g * 16, 16)]
                    fidx_v[buf, i, ds, pl.ds(g * 16, 16)] = pv + base

    def ent_tiles(out_hbm, m):
        build_fidx(0, 0)
        for i in range(2):
            for ds in range(8):
                pltpu.async_copy(ef_hbm.at[fidx_v.at[0, i, ds]],
                                 tile_v.at[0, i, ds], gsem)

        def j_body(j, _):
            buf = j % 2

            @pl.when(j > 0)
            def _():
                for i in range(2):
                    pltpu.make_async_copy(tile_v.at[1 - buf, i],
                                          out_hbm.at[h, m, i, j - 1], osem).wait()

            @pl.when(j < 7)
            def _():
                build_fidx(j + 1, 1 - buf)
                for i in range(2):
                    for ds in range(8):
                        pltpu.async_copy(ef_hbm.at[fidx_v.at[1 - buf, i, ds]],
                                         tile_v.at[1 - buf, i, ds], gsem)

            for i in range(2):
                for ds in range(8):
                    pltpu.make_async_copy(ef_hbm.at[fidx_v.at[buf, i, ds]],
                                          tile_v.at[buf, i, ds], gsem).wait()
            for i in range(2):
                pltpu.async_copy(tile_v.at[buf, i], out_hbm.at[h, m, i, j], osem)
            return 0

        lax.fori_loop(0, 8, j_body, 0, unroll=False)
        for i in range(2):
            pltpu.make_async_copy(tile_v.at[1, i], out_hbm.at[h, m, i, 7],
                                  osem).wait()

    for mm in range(2):
        m = m2 + mm
        pltpu.sync_copy(hphi_hbm.at[h, m], eidx_v)
        build_pidx(64)
        ent_tiles(hsp_hbm, m)
        pltpu.sync_copy(tphi_hbm.at[h, m], eidx_v)
        build_pidx(64)
        ent_tiles(tsp_hbm, m)

    @pl.when(wid < 16)
    def _():
        i = wid // 8
        j = wid % 8
        pltpu.sync_copy(v_hbm.at[pl.ds(j * 128, 128)], eidx_v.at[pl.ds(0, 128)])
        build_pidx(8)
        for ds in range(8):
            base = i * 8000512 + ds * 128
            for g in range(8):
                pv = pidx_v[pl.ds(g * 16, 16)]
                fidx_v[0, 0, ds, pl.ds(g * 16, 16)] = pv + base
        for ds in range(8):
            pltpu.async_copy(ef_hbm.at[fidx_v.at[0, 0, ds]], tile_v.at[0, 0, ds], gsem)
        for ds in range(8):
            pltpu.make_async_copy(ef_hbm.at[fidx_v.at[0, 0, ds]],
                                  tile_v.at[0, 0, ds], gsem).wait()
        pltpu.sync_copy(tile_v.at[0, 0], vsp_hbm.at[i, j])


def _sc_call():
    if "sc" not in _cache:
        mesh = plsc.VectorSubcoreMesh(core_axis_name="c", subcore_axis_name="s",
                                      num_cores=NC, num_subcores=NS)
        _cache["sc"] = pl.kernel(
            _sc_body,
            out_type=(
                jax.ShapeDtypeStruct((H, M, 2, 8, 8, 128), jnp.float32),
                jax.ShapeDtypeStruct((H, M, 2, 8, 8, 128), jnp.float32),
                jax.ShapeDtypeStruct((2, 8, 8, 128), jnp.float32),
            ),
            mesh=mesh,
            scratch_types=[
                pltpu.VMEM((B,), jnp.int32),
                pltpu.VMEM((B,), jnp.int32),
                pltpu.VMEM((2, 2, 8, 128), jnp.int32),
                pltpu.VMEM((2, 2, 8, 128), jnp.float32),
                pltpu.SemaphoreType.DMA,
                pltpu.SemaphoreType.DMA,
            ],
            compiler_params=pltpu.CompilerParams(needs_layout_passes=False,
                                                 use_tc_tiling_on_sc=False),
        )
    return _cache["sc"]


def kernel(h_i, R_i, t_i, v_i, entity_emb, relation_emb):
    eT = entity_emb.T
    relT = relation_emb.reshape(NR, 256).T
    hphi = jnp.transpose(h_i, (1, 2, 0))
    tphi = jnp.transpose(t_i, (1, 2, 0))
    rphi = jnp.transpose(R_i, (1, 2, 0))

    ef = _detile()(eT).reshape(-1)
    rsp = _rs_call()(relT, rphi.reshape(H, M, 1, B))
    hsp, tsp, vsp = _sc_call()(ef, hphi, tphi, v_i)

    hs = (hsp.transpose(0, 1, 3, 5, 2, 4)
             .reshape(H, M, B, 16)
             .transpose(2, 0, 1, 3))
    ts = (tsp.transpose(0, 1, 3, 5, 2, 4)
             .reshape(H, M, B, 16)
             .transpose(2, 0, 1, 3))
    Rs = (rsp.reshape(H, M, 16, 16, B)
             .transpose(4, 0, 1, 2, 3))
    vs = (vsp.transpose(1, 3, 0, 2)
             .reshape(B, 16))
    return (hs, Rs, ts, vs)
